# baseline (speedup 1.0000x reference)
import jax
import jax.numpy as jnp
from jax.experimental import pallas as pl
from jax.experimental.pallas import tpu as pltpu

_NBUF = 4


def _body(ptr_ref, keys_ref, idx_hbm, lab_hbm, buf_hbm, midx_hbm, mlab_hbm,
          outb_hbm, outi_hbm, outl_hbm,
          ring, keysT_v, idx_v, lab_v, gsems, ssems, sem_i, sem_l):
    f, K = buf_hbm.shape
    B = keys_ref.shape[0]
    CC = ring.shape[2]
    nchunks = K // CC
    p = ptr_ref[0]
    p = jnp.clip(p, 0, K - B)
    p = pl.multiple_of(p, 128)
    pslab = p // B

    def gather(i, b):
        return pltpu.make_async_copy(
            buf_hbm.at[:, pl.ds(i * CC, CC)], ring.at[b], gsems.at[b])

    def scatter(i, b):
        for h in range(CC // B):
            g = i * (CC // B) + h
            dst = outb_hbm.at[:, pl.ds(g * B, B)]

            @pl.when(g == pslab)
            def _():
                pltpu.make_async_copy(keysT_v, dst, ssems.at[b]).start()

            @pl.when(g != pslab)
            def _():
                pltpu.make_async_copy(
                    ring.at[b, :, pl.ds(h * B, B)], dst, ssems.at[b]).start()

        return pltpu.make_async_copy(
            ring.at[b], outb_hbm.at[:, pl.ds(i * CC, CC)], ssems.at[b])

    gi = pltpu.make_async_copy(midx_hbm, idx_v, sem_i)
    gi.start()
    gl = pltpu.make_async_copy(mlab_hbm, lab_v, sem_l)
    gl.start()

    h_g = [None] * nchunks
    h_s = [None] * nchunks
    lag = _NBUF - 1
    for i in range(nchunks):
        b = i % _NBUF
        if i >= _NBUF:
            h_s[i - _NBUF].wait()
        h_g[i] = gather(i, b)
        h_g[i].start()
        if i == 0:
            keysT_v[...] = keys_ref[...].T
        k = i - lag
        if k >= 0:
            h_g[k].wait()
            h_s[k] = scatter(k, k % _NBUF)
    for k in range(max(nchunks - lag, 0), nchunks):
        h_g[k].wait()
        h_s[k] = scatter(k, k % _NBUF)
    for k in range(max(nchunks - _NBUF, 0), nchunks):
        h_s[k].wait()

    gi.wait()
    gl.wait()
    si = pltpu.make_async_copy(idx_hbm, idx_v.at[pl.ds(p, B)], sem_i)
    si.start()
    sl = pltpu.make_async_copy(lab_hbm, lab_v.at[pl.ds(p, B)], sem_l)
    sl.start()
    si.wait()
    sl.wait()
    so_i = pltpu.make_async_copy(idx_v, outi_hbm, sem_i)
    so_i.start()
    so_l = pltpu.make_async_copy(lab_v, outl_hbm, sem_l)
    so_l.start()
    so_i.wait()
    so_l.wait()


def kernel(keys, indices, labels, buffer, mem_indices, mem_labels, ptr,
           num_updates):
    f, K = buffer.shape
    B = keys.shape[0]
    CC = 16384

    new_buffer, new_indices, new_labels = pl.pallas_call(
        _body,
        in_specs=[
            pl.BlockSpec(memory_space=pltpu.SMEM),
            pl.BlockSpec(memory_space=pltpu.VMEM),
            pl.BlockSpec(memory_space=pl.MemorySpace.ANY),
            pl.BlockSpec(memory_space=pl.MemorySpace.ANY),
            pl.BlockSpec(memory_space=pl.MemorySpace.ANY),
            pl.BlockSpec(memory_space=pl.MemorySpace.ANY),
            pl.BlockSpec(memory_space=pl.MemorySpace.ANY),
        ],
        out_specs=[
            pl.BlockSpec(memory_space=pl.MemorySpace.ANY),
            pl.BlockSpec(memory_space=pl.MemorySpace.ANY),
            pl.BlockSpec(memory_space=pl.MemorySpace.ANY),
        ],
        out_shape=[
            jax.ShapeDtypeStruct((f, K), buffer.dtype),
            jax.ShapeDtypeStruct((K,), mem_indices.dtype),
            jax.ShapeDtypeStruct((K,), mem_labels.dtype),
        ],
        scratch_shapes=[
            pltpu.VMEM((_NBUF, f, CC), jnp.float32),
            pltpu.VMEM((f, B), keys.dtype),
            pltpu.VMEM((K,), mem_indices.dtype),
            pltpu.VMEM((K,), mem_labels.dtype),
            pltpu.SemaphoreType.DMA((_NBUF,)),
            pltpu.SemaphoreType.DMA((_NBUF,)),
            pltpu.SemaphoreType.DMA,
            pltpu.SemaphoreType.DMA,
        ],
    )(ptr, keys, indices, labels, buffer, mem_indices, mem_labels)

    p = ptr[0]
    is_reliable = (p + B) >= K
    new_ptr = jnp.reshape(((p + B) % K).astype(ptr.dtype), (1,))
    new_num_updates = num_updates + 1
    return (new_buffer, new_indices, new_labels, new_ptr, new_num_updates,
            is_reliable)


# speedup vs baseline: 1.0113x; 1.0113x over previous
import jax
import jax.numpy as jnp
from jax.experimental import pallas as pl
from jax.experimental.pallas import tpu as pltpu

_NBUF = 6


def _body(ptr_ref, keys_ref, idx_hbm, lab_hbm, buf_hbm, midx_hbm, mlab_hbm,
          outb_hbm, outi_hbm, outl_hbm,
          ring, keysT_v, idx_v, lab_v, gsems, ssems, sem_i, sem_l):
    f, K = buf_hbm.shape
    B = keys_ref.shape[0]
    CC = ring.shape[2]
    nchunks = K // CC
    p = ptr_ref[0]
    p = jnp.clip(p, 0, K - B)
    p = pl.multiple_of(p, 128)
    pslab = p // B

    def gather(i, b):
        return pltpu.make_async_copy(
            buf_hbm.at[:, pl.ds(i * CC, CC)], ring.at[b], gsems.at[b])

    def scatter(i, b):
        for h in range(CC // B):
            g = i * (CC // B) + h
            dst = outb_hbm.at[:, pl.ds(g * B, B)]

            @pl.when(g == pslab)
            def _():
                pltpu.make_async_copy(keysT_v, dst, ssems.at[b]).start()

            @pl.when(g != pslab)
            def _():
                pltpu.make_async_copy(
                    ring.at[b, :, pl.ds(h * B, B)], dst, ssems.at[b]).start()

        return pltpu.make_async_copy(
            ring.at[b], outb_hbm.at[:, pl.ds(i * CC, CC)], ssems.at[b])

    gi = pltpu.make_async_copy(midx_hbm, idx_v, sem_i)
    gi.start()
    gl = pltpu.make_async_copy(mlab_hbm, lab_v, sem_l)
    gl.start()

    h_g = [None] * nchunks
    h_s = [None] * nchunks
    lag = _NBUF - 1
    for i in range(nchunks):
        b = i % _NBUF
        if i >= _NBUF:
            h_s[i - _NBUF].wait()
        h_g[i] = gather(i, b)
        h_g[i].start()
        if i == 0:
            keysT_v[...] = keys_ref[...].T
        k = i - lag
        if k >= 0:
            h_g[k].wait()
            h_s[k] = scatter(k, k % _NBUF)
    for k in range(max(nchunks - lag, 0), nchunks):
        h_g[k].wait()
        h_s[k] = scatter(k, k % _NBUF)
    for k in range(max(nchunks - _NBUF, 0), nchunks):
        h_s[k].wait()

    gi.wait()
    gl.wait()
    si = pltpu.make_async_copy(idx_hbm, idx_v.at[pl.ds(p, B)], sem_i)
    si.start()
    sl = pltpu.make_async_copy(lab_hbm, lab_v.at[pl.ds(p, B)], sem_l)
    sl.start()
    si.wait()
    sl.wait()
    so_i = pltpu.make_async_copy(idx_v, outi_hbm, sem_i)
    so_i.start()
    so_l = pltpu.make_async_copy(lab_v, outl_hbm, sem_l)
    so_l.start()
    so_i.wait()
    so_l.wait()


def kernel(keys, indices, labels, buffer, mem_indices, mem_labels, ptr,
           num_updates):
    f, K = buffer.shape
    B = keys.shape[0]
    CC = 8192

    new_buffer, new_indices, new_labels = pl.pallas_call(
        _body,
        in_specs=[
            pl.BlockSpec(memory_space=pltpu.SMEM),
            pl.BlockSpec(memory_space=pltpu.VMEM),
            pl.BlockSpec(memory_space=pl.MemorySpace.ANY),
            pl.BlockSpec(memory_space=pl.MemorySpace.ANY),
            pl.BlockSpec(memory_space=pl.MemorySpace.ANY),
            pl.BlockSpec(memory_space=pl.MemorySpace.ANY),
            pl.BlockSpec(memory_space=pl.MemorySpace.ANY),
        ],
        out_specs=[
            pl.BlockSpec(memory_space=pl.MemorySpace.ANY),
            pl.BlockSpec(memory_space=pl.MemorySpace.ANY),
            pl.BlockSpec(memory_space=pl.MemorySpace.ANY),
        ],
        out_shape=[
            jax.ShapeDtypeStruct((f, K), buffer.dtype),
            jax.ShapeDtypeStruct((K,), mem_indices.dtype),
            jax.ShapeDtypeStruct((K,), mem_labels.dtype),
        ],
        scratch_shapes=[
            pltpu.VMEM((_NBUF, f, CC), jnp.float32),
            pltpu.VMEM((f, B), keys.dtype),
            pltpu.VMEM((K,), mem_indices.dtype),
            pltpu.VMEM((K,), mem_labels.dtype),
            pltpu.SemaphoreType.DMA((_NBUF,)),
            pltpu.SemaphoreType.DMA((_NBUF,)),
            pltpu.SemaphoreType.DMA,
            pltpu.SemaphoreType.DMA,
        ],
    )(ptr, keys, indices, labels, buffer, mem_indices, mem_labels)

    p = ptr[0]
    is_reliable = (p + B) >= K
    new_ptr = jnp.reshape(((p + B) % K).astype(ptr.dtype), (1,))
    new_num_updates = num_updates + 1
    return (new_buffer, new_indices, new_labels, new_ptr, new_num_updates,
            is_reliable)


# speedup vs baseline: 1.0780x; 1.0660x over previous
import jax
import jax.numpy as jnp
from jax.experimental import pallas as pl
from jax.experimental.pallas import tpu as pltpu

_NBUF = 6


def _body(ptr_ref, keys_ref, idx_hbm, lab_hbm, buf_hbm, midx_hbm, mlab_hbm,
          outb_hbm, outi_hbm, outl_hbm,
          ring, keysT_v, idx_v, lab_v, gsems, ssems, sem_i, sem_l):
    f, K = buf_hbm.shape
    B = keys_ref.shape[0]
    CC = ring.shape[2]
    nchunks = K // CC
    p = ptr_ref[0]
    p = jnp.clip(p, 0, K - B)
    p = pl.multiple_of(p, 128)
    pslab = p // B

    def gather(i, b):
        return pltpu.make_async_copy(
            buf_hbm.at[:, pl.ds(i * CC, CC)], ring.at[b], gsems.at[b])

    def scatter(i, b):
        for h in range(CC // B):
            g = i * (CC // B) + h
            dst = outb_hbm.at[:, pl.ds(g * B, B)]

            @pl.when(g == pslab)
            def _():
                pltpu.make_async_copy(keysT_v, dst, ssems.at[b]).start()

            @pl.when(g != pslab)
            def _():
                pltpu.make_async_copy(
                    ring.at[b, :, pl.ds(h * B, B)], dst, ssems.at[b]).start()

        return pltpu.make_async_copy(
            ring.at[b], outb_hbm.at[:, pl.ds(i * CC, CC)], ssems.at[b])

    gi = pltpu.make_async_copy(midx_hbm, idx_v, sem_i)
    gi.start()
    gl = pltpu.make_async_copy(mlab_hbm, lab_v, sem_l)
    gl.start()

    h_g = [None] * nchunks
    h_s = [None] * nchunks
    lag = _NBUF - 1
    for i in range(nchunks):
        b = i % _NBUF
        if i >= _NBUF:
            h_s[i - _NBUF].wait()
        h_g[i] = gather(i, b)
        h_g[i].start()
        if i == 0:
            keysT_v[...] = keys_ref[...].T
        if i == 1:
            gi.wait()
            gl.wait()
            si = pltpu.make_async_copy(idx_hbm, idx_v.at[pl.ds(p, B)], sem_i)
            si.start()
            sl = pltpu.make_async_copy(lab_hbm, lab_v.at[pl.ds(p, B)], sem_l)
            sl.start()
        if i == 2:
            si.wait()
            sl.wait()
            so_i = pltpu.make_async_copy(idx_v, outi_hbm, sem_i)
            so_i.start()
            so_l = pltpu.make_async_copy(lab_v, outl_hbm, sem_l)
            so_l.start()
        k = i - lag
        if k >= 0:
            h_g[k].wait()
            h_s[k] = scatter(k, k % _NBUF)
    for k in range(max(nchunks - lag, 0), nchunks):
        h_g[k].wait()
        h_s[k] = scatter(k, k % _NBUF)
    for k in range(max(nchunks - _NBUF, 0), nchunks):
        h_s[k].wait()

    so_i.wait()
    so_l.wait()


def kernel(keys, indices, labels, buffer, mem_indices, mem_labels, ptr,
           num_updates):
    f, K = buffer.shape
    B = keys.shape[0]
    CC = 8192

    new_buffer, new_indices, new_labels = pl.pallas_call(
        _body,
        in_specs=[
            pl.BlockSpec(memory_space=pltpu.SMEM),
            pl.BlockSpec(memory_space=pltpu.VMEM),
            pl.BlockSpec(memory_space=pl.MemorySpace.ANY),
            pl.BlockSpec(memory_space=pl.MemorySpace.ANY),
            pl.BlockSpec(memory_space=pl.MemorySpace.ANY),
            pl.BlockSpec(memory_space=pl.MemorySpace.ANY),
            pl.BlockSpec(memory_space=pl.MemorySpace.ANY),
        ],
        out_specs=[
            pl.BlockSpec(memory_space=pl.MemorySpace.ANY),
            pl.BlockSpec(memory_space=pl.MemorySpace.ANY),
            pl.BlockSpec(memory_space=pl.MemorySpace.ANY),
        ],
        out_shape=[
            jax.ShapeDtypeStruct((f, K), buffer.dtype),
            jax.ShapeDtypeStruct((K,), mem_indices.dtype),
            jax.ShapeDtypeStruct((K,), mem_labels.dtype),
        ],
        scratch_shapes=[
            pltpu.VMEM((_NBUF, f, CC), jnp.float32),
            pltpu.VMEM((f, B), keys.dtype),
            pltpu.VMEM((K,), mem_indices.dtype),
            pltpu.VMEM((K,), mem_labels.dtype),
            pltpu.SemaphoreType.DMA((_NBUF,)),
            pltpu.SemaphoreType.DMA((_NBUF,)),
            pltpu.SemaphoreType.DMA,
            pltpu.SemaphoreType.DMA,
        ],
    )(ptr, keys, indices, labels, buffer, mem_indices, mem_labels)

    p = ptr[0]
    is_reliable = (p + B) >= K
    new_ptr = jnp.reshape(((p + B) % K).astype(ptr.dtype), (1,))
    new_num_updates = num_updates + 1
    return (new_buffer, new_indices, new_labels, new_ptr, new_num_updates,
            is_reliable)
